# baseline (speedup 1.0000x reference)
import functools

import jax
import jax.numpy as jnp
from jax import lax
from jax.experimental import pallas as pl
from jax.experimental.pallas import tpu as pltpu
from jax.experimental.pallas import tpu_sc as plsc

_SEQ = 200
_BATCH = 4096
_DIM = 64
_VOCAB = 1000000
_NC = 2
_NS = 16
_NW = _NC * _NS
_BTILE = _BATCH // _NW
_LANES = 16
_BGROUPS = _BTILE // _LANES
_RSTRIDE = _DIM + 1


def _make_sc_call():
    mesh = plsc.VectorSubcoreMesh(
        core_axis_name="c", subcore_axis_name="s",
        num_cores=_NC, num_subcores=_NS)

    @functools.partial(
        pl.kernel,
        out_type=jax.ShapeDtypeStruct(
            (_SEQ, _DIM // 8, _NW, 8, _BTILE), jnp.float32),
        mesh=mesh,
        scratch_types=[
            pltpu.VMEM((2, _BTILE), jnp.int32),
            pltpu.VMEM((_BTILE, 2 * _DIM), jnp.float32),
            pltpu.VMEM((_BTILE, 2 * _DIM), jnp.float32),
            pltpu.VMEM((_BTILE * _RSTRIDE,), jnp.float32),
            pltpu.VMEM((2, _DIM // 8, 8, _BTILE), jnp.float32),
            pltpu.VMEM((_SEQ, _DIM), jnp.float32),
            pltpu.SemaphoreType.DMA((2,)),
            pltpu.SemaphoreType.DMA((2,)),
        ],
        compiler_params=pltpu.CompilerParams(
            use_tc_tiling_on_sc=True, needs_layout_passes=False),
    )
    def sc_embed(xt_hbm, tok_hbm, pos_hbm, out_hbm, idx_v, rows_a, rows_b,
                 rp_v, xp_v, pos_v, gsem, osem):
        w = lax.axis_index("s") * _NC + lax.axis_index("c")
        b0 = w * _BTILE
        pltpu.sync_copy(pos_hbm, pos_v)

        iota = lax.iota(jnp.int32, _LANES)
        iota_rs = iota * _RSTRIDE

        def stage_and_fire(s, p, rows):
            pltpu.sync_copy(xt_hbm.at[s, pl.ds(b0, _BTILE)], idx_v.at[p])
            pltpu.async_copy(tok_hbm.at[idx_v.at[p]], rows, gsem.at[p])

        def process(s, p, rows):
            pvec = [pos_v[s, pl.ds(c * _LANES, _LANES)] for c in range(4)]

            @plsc.parallel_loop(0, _BTILE, 1, unroll=4)
            def _(b):
                base = b * _RSTRIDE
                for c in range(4):
                    v = rows[b, pl.ds(c * _LANES, _LANES)] + pvec[c]
                    rp_v[pl.ds(base + c * _LANES, _LANES)] = v

            @plsc.parallel_loop(0, _DIM * _BGROUPS, 1, unroll=8)
            def _(k):
                d = k // _BGROUPS
                g = k - d * _BGROUPS
                flat = iota_rs + (g * _LANES * _RSTRIDE + d)
                vals = plsc.load_gather(rp_v, [flat])
                xp_v[p, d // 8, d % 8, pl.ds(g * _LANES, _LANES)] = vals

            pltpu.async_copy(xp_v.at[p], out_hbm.at[s, :, w], osem.at[p])

        stage_and_fire(0, 0, rows_a)

        def body(s, carry):
            p = lax.rem(s, 2)
            q = 1 - p

            @pl.when(s + 1 < _SEQ)
            def _():
                @pl.when(s >= 1)
                def _():
                    pltpu.make_async_copy(
                        xp_v.at[q], out_hbm.at[0, :, w], osem.at[q]).wait()

                @pl.when(p == 0)
                def _():
                    stage_and_fire(s + 1, 1, rows_b)

                @pl.when(p == 1)
                def _():
                    stage_and_fire(s + 1, 0, rows_a)

            @pl.when(p == 0)
            def _():
                pltpu.make_async_copy(tok_hbm.at[idx_v.at[0]], rows_a,
                                      gsem.at[0]).wait()
                process(s, 0, rows_a)

            @pl.when(p == 1)
            def _():
                pltpu.make_async_copy(tok_hbm.at[idx_v.at[1]], rows_b,
                                      gsem.at[1]).wait()
                process(s, 1, rows_b)

            return carry

        lax.fori_loop(0, _SEQ, body, 0)

        for p in range(2):
            pltpu.make_async_copy(xp_v.at[p], out_hbm.at[0, :, w],
                                  osem.at[p]).wait()

    return sc_embed


_sc_embed = _make_sc_call()

_VTILES = _VOCAB // 128
_VTAIL = _VOCAB - _VTILES * 128
_TITER = _VTILES // _NW + 1
_TSTRIDE = 129


def _make_transpose_call():
    mesh = plsc.VectorSubcoreMesh(
        core_axis_name="c", subcore_axis_name="s",
        num_cores=_NC, num_subcores=_NS)

    @functools.partial(
        pl.kernel,
        out_type=jax.ShapeDtypeStruct(((_VTILES + 1) * 128, 2 * _DIM),
                                      jnp.float32),
        mesh=mesh,
        scratch_types=[
            pltpu.VMEM((2, _DIM, 128), jnp.float32),
            pltpu.VMEM((2, 128, _TSTRIDE), jnp.float32),
            pltpu.SemaphoreType.DMA((2,)),
            pltpu.SemaphoreType.DMA((2,)),
        ],
        compiler_params=pltpu.CompilerParams(
            use_tc_tiling_on_sc=True, needs_layout_passes=False),
    )
    def sc_tpose(tt_hbm, out_hbm, in_v, op_v, rsem, wsem):
        w = lax.axis_index("s") * _NC + lax.axis_index("c")
        iota = lax.iota(jnp.int32, _LANES)
        iota_ts = iota * _TSTRIDE

        def fire(i, p):
            tv = i * _NW + w
            pltpu.async_copy(tt_hbm.at[:, pl.ds(tv * 128, 128)],
                             in_v.at[p], rsem.at[p])

        fire(0, 0)

        def body(i, carry):
            p = lax.rem(i, 2)
            q = 1 - p
            tv = i * _NW + w
            pb = jnp.broadcast_to(p, (_LANES,)).astype(jnp.int32)

            @pl.when((i >= 2) & ((i - 2) * _NW + w <= _VTILES))
            def _():
                pltpu.make_async_copy(op_v.at[0, :, pl.ds(0, 128)],
                                      out_hbm.at[pl.ds(0, 128)],
                                      wsem.at[p]).wait()

            @pl.when((i + 1) * _NW + w <= _VTILES)
            def _():
                fire(i + 1, q)

            @pl.when(tv <= _VTILES)
            def _():
                pltpu.make_async_copy(tt_hbm.at[:, pl.ds(0, 128)],
                                      in_v.at[p], rsem.at[p]).wait()

                @plsc.parallel_loop(0, _DIM * 8, 1, unroll=8)
                def _(k):
                    d = k // 8
                    vg = k - d * 8
                    vals = in_v[p, d, pl.ds(vg * _LANES, _LANES)]
                    row = iota + vg * _LANES
                    col = jnp.broadcast_to(d, (_LANES,)).astype(jnp.int32)
                    plsc.store_scatter(op_v, [pb, row, col], vals)

                pltpu.async_copy(op_v.at[p, :, pl.ds(0, 128)],
                                 out_hbm.at[pl.ds(tv * 128, 128)],
                                 wsem.at[p])

            return carry

        lax.fori_loop(0, _TITER, body, 0)

        for j in (_TITER - 2, _TITER - 1):
            @pl.when(j * _NW + w <= _VTILES)
            def _():
                pltpu.make_async_copy(op_v.at[0, :, pl.ds(0, 128)],
                                      out_hbm.at[pl.ds(0, 128)],
                                      wsem.at[j % 2]).wait()

    return sc_tpose


_sc_tpose = _make_transpose_call()


@jax.jit
def kernel(x, token_table, pos_table):
    tok128 = _sc_tpose(token_table.T)
    out5 = _sc_embed(x.T, tok128, pos_table)
    return out5.transpose(2, 4, 0, 1, 3).reshape(_BATCH, _SEQ, _DIM)


# speedup vs baseline: 1.7243x; 1.7243x over previous
import functools

import jax
import jax.numpy as jnp
from jax import lax
from jax.experimental import pallas as pl
from jax.experimental.pallas import tpu as pltpu
from jax.experimental.pallas import tpu_sc as plsc

_SEQ = 200
_BATCH = 4096
_DIM = 64
_VOCAB = 1000000
_NC = 2
_NS = 16
_NW = _NC * _NS
_BTILE = _BATCH // _NW
_LANES = 16
_BGROUPS = _BTILE // _LANES
_RSTRIDE = _DIM + 1


def _make_sc_call():
    mesh = plsc.VectorSubcoreMesh(
        core_axis_name="c", subcore_axis_name="s",
        num_cores=_NC, num_subcores=_NS)

    @functools.partial(
        pl.kernel,
        out_type=jax.ShapeDtypeStruct(
            (_SEQ, _DIM // 8, _NW, 8, _BTILE), jnp.float32),
        mesh=mesh,
        scratch_types=[
            pltpu.VMEM((2, _BTILE), jnp.int32),
            pltpu.VMEM((_BTILE, _DIM), jnp.float32),
            pltpu.VMEM((_BTILE, _DIM), jnp.float32),
            pltpu.VMEM((_BTILE * _RSTRIDE,), jnp.float32),
            pltpu.VMEM((2, _DIM // 8, 8, _BTILE), jnp.float32),
            pltpu.VMEM((_SEQ, _DIM), jnp.float32),
            pltpu.SemaphoreType.DMA((2,)),
            pltpu.SemaphoreType.DMA((2,)),
        ],
        compiler_params=pltpu.CompilerParams(
            use_tc_tiling_on_sc=False, needs_layout_passes=False),
    )
    def sc_embed(xt_hbm, tok_hbm, pos_hbm, out_hbm, idx_v, rows_a, rows_b,
                 rp_v, xp_v, pos_v, gsem, osem):
        w = lax.axis_index("s") * _NC + lax.axis_index("c")
        b0 = w * _BTILE
        pltpu.sync_copy(pos_hbm, pos_v)

        iota = lax.iota(jnp.int32, _LANES)
        iota_rs = iota * _RSTRIDE

        def stage_and_fire(s, p, rows):
            pltpu.sync_copy(xt_hbm.at[s, pl.ds(b0, _BTILE)], idx_v.at[p])

            @plsc.parallel_loop(0, _BGROUPS, 1, unroll=4)
            def _(g):
                sl = pl.ds(g * _LANES, _LANES)
                idx_v[p, sl] = idx_v[p, sl] * 2

            pltpu.async_copy(tok_hbm.at[idx_v.at[p]], rows, gsem.at[p])

        def process(s, p, rows):
            pvec = [pos_v[s, pl.ds(c * _LANES, _LANES)] for c in range(4)]

            @plsc.parallel_loop(0, _BTILE, 1, unroll=4)
            def _(b):
                base = b * _RSTRIDE
                for c in range(4):
                    v = rows[b, pl.ds(c * _LANES, _LANES)] + pvec[c]
                    rp_v[pl.ds(base + c * _LANES, _LANES)] = v

            @plsc.parallel_loop(0, _DIM * _BGROUPS, 1, unroll=8)
            def _(k):
                d = k // _BGROUPS
                g = k - d * _BGROUPS
                flat = iota_rs + (g * _LANES * _RSTRIDE + d)
                vals = plsc.load_gather(rp_v, [flat])
                xp_v[p, d // 8, d % 8, pl.ds(g * _LANES, _LANES)] = vals

            pltpu.async_copy(xp_v.at[p], out_hbm.at[s, :, w], osem.at[p])

        stage_and_fire(0, 0, rows_a)

        def body(s, carry):
            p = lax.rem(s, 2)
            q = 1 - p

            @pl.when(s + 1 < _SEQ)
            def _():
                @pl.when(s >= 1)
                def _():
                    pltpu.make_async_copy(
                        xp_v.at[q], out_hbm.at[0, :, w], osem.at[q]).wait()

                @pl.when(p == 0)
                def _():
                    stage_and_fire(s + 1, 1, rows_b)

                @pl.when(p == 1)
                def _():
                    stage_and_fire(s + 1, 0, rows_a)

            @pl.when(p == 0)
            def _():
                pltpu.make_async_copy(tok_hbm.at[idx_v.at[0]], rows_a,
                                      gsem.at[0]).wait()
                process(s, 0, rows_a)

            @pl.when(p == 1)
            def _():
                pltpu.make_async_copy(tok_hbm.at[idx_v.at[1]], rows_b,
                                      gsem.at[1]).wait()
                process(s, 1, rows_b)

            return carry

        lax.fori_loop(0, _SEQ, body, 0)

        for p in range(2):
            pltpu.make_async_copy(xp_v.at[p], out_hbm.at[0, :, w],
                                  osem.at[p]).wait()

    return sc_embed


_sc_embed = _make_sc_call()


@jax.jit
def kernel(x, token_table, pos_table):
    tok2 = jnp.pad(token_table, ((0, 0), (0, _DIM))).reshape(2 * _VOCAB, _DIM)
    out5 = _sc_embed(x.T, tok2, pos_table)
    return out5.transpose(2, 4, 0, 1, 3).reshape(_BATCH, _SEQ, _DIM)
